# Initial kernel scaffold; baseline (speedup 1.0000x reference)
import jax
import jax.numpy as jnp
from jax.experimental import pallas as pl


def kernel(x, edge_index, edge_attr, batch, params):
    raise NotImplementedError("write your pallas kernel here")


# speedup vs baseline: 10.4827x; 10.4827x over previous
import functools
import jax
import jax.numpy as jnp
from jax import lax
from jax.experimental import pallas as pl
from jax.experimental.pallas import tpu as pltpu
from jax.experimental.pallas import tpu_sc as plsc

N = 10000
NP = 10240
E = 320000
H = 128
G = 64
RB = 1024
EB = 1280
NC, NS = 2, 16
NW = NC * NS
EPW = E // NW
CH = 40
NCHUNK = EPW // CH
RPT = NP // NS
NPW = NP // NW
NSC = NPW // CH
NPACK = NP // 8

f32 = jnp.float32
i32 = jnp.int32

_SC_PARAMS = pltpu.CompilerParams(
    needs_layout_passes=False, use_tc_tiling_on_sc=False)


def _a1_body(x_ref, wn_ref, bn_ref, wlin_ref, att_ref,
             x0_ref, xs_ref, nsc_ref):
    x0 = jnp.maximum(x_ref[...] @ wn_ref[...] + bn_ref[...], 0.0)
    xs = x0 @ wlin_ref[...]
    x0_ref[...] = x0
    xs_ref[...] = xs
    a = lax.dot_general(att_ref[...], xs, (((1,), (1,)), ((), ())))
    nsc_ref[...] = jnp.concatenate([a, jnp.zeros((6, RB), f32)], axis=0)


def _a1(x, wn, bn, wlin, att):
    return pl.pallas_call(
        _a1_body,
        grid=(NP // RB,),
        in_specs=[
            pl.BlockSpec((RB, H), lambda j: (j, 0)),
            pl.BlockSpec((H, H), lambda j: (0, 0)),
            pl.BlockSpec((1, H), lambda j: (0, 0)),
            pl.BlockSpec((H, H), lambda j: (0, 0)),
            pl.BlockSpec((2, H), lambda j: (0, 0)),
        ],
        out_specs=[
            pl.BlockSpec((RB, H), lambda j: (j, 0)),
            pl.BlockSpec((RB, H), lambda j: (j, 0)),
            pl.BlockSpec((8, RB), lambda j: (0, j)),
        ],
        out_shape=[
            jax.ShapeDtypeStruct((NP, H), f32),
            jax.ShapeDtypeStruct((NP, H), f32),
            jax.ShapeDtypeStruct((8, NP), f32),
        ],
    )(x, wn, bn, wlin, att)


def _a2_body(ea_ref, we_ref, be_ref, wedge_ref, attedge_ref,
             s16_ref, s3_ref):
    eap = jnp.maximum(ea_ref[...] @ we_ref[...] + be_ref[...], 0.0)
    w = wedge_ref[...]
    ae = attedge_ref[...]
    v1 = w[0:H, :] @ ae[0, :][:, None]
    v2 = w[H:2 * H, :] @ ae[1, :][:, None]
    v3 = w[2 * H:, :] @ ae[2, :][:, None]
    s = eap @ jnp.concatenate([v1, v2, v3], axis=1)
    ones = jnp.ones((EB, 1), f32)
    s16_ref[...] = jnp.concatenate(
        [s, ones, jnp.zeros((EB, 12), f32)], axis=1)
    st = lax.dot_general(jnp.eye(8, 3, dtype=f32), s,
                         (((1,), (1,)), ((), ())))
    s3_ref[...] = st


def _a2(edge_attr, we, be, wedge3, attedge3):
    de = edge_attr.shape[1]
    return pl.pallas_call(
        _a2_body,
        grid=(E // EB,),
        in_specs=[
            pl.BlockSpec((EB, de), lambda j: (j, 0)),
            pl.BlockSpec((de, H), lambda j: (0, 0)),
            pl.BlockSpec((1, H), lambda j: (0, 0)),
            pl.BlockSpec((3 * H, H), lambda j: (0, 0)),
            pl.BlockSpec((3, H), lambda j: (0, 0)),
        ],
        out_specs=[
            pl.BlockSpec((EB, 16), lambda j: (j, 0)),
            pl.BlockSpec((8, EB), lambda j: (0, j)),
        ],
        out_shape=[
            jax.ShapeDtypeStruct((E, 16), f32),
            jax.ShapeDtypeStruct((8, E), f32),
        ],
    )(edge_attr, we, be, wedge3, attedge3)


def _p0_body(dst_hbm, s16_hbm, degp_hbm, acc_sh, dstb, dstb3, rowsb,
             stage, sem):
    sid = lax.axis_index("s")
    cid = lax.axis_index("c")
    wid = sid * NC + cid

    def _zrow(r, _):
        for q in range(8):
            stage[r, pl.ds(q * 16, 16)] = jnp.zeros((16,), f32)
        return _
    lax.fori_loop(0, CH, _zrow, None)
    ppw = NPACK // NW
    for k in range(ppw // CH):
        pltpu.sync_copy(stage, acc_sh.at[pl.ds(wid * ppw + k * CH, CH)])
    plsc.subcore_barrier()

    def _chunk(c, _):
        eb = wid * EPW + c * CH
        pltpu.sync_copy(dst_hbm.at[pl.ds(eb, CH)], dstb)
        pltpu.sync_copy(s16_hbm.at[pl.ds(eb, CH)], rowsb)
        for off, j0 in ((0, 0), (16, 0), (24, 8)):
            dv = dstb[pl.ds(off, 16)]
            dstb3[pl.ds(off, 16)] = lax.shift_right_logical(dv, 3)
            for j in range(j0, 16):
                e = off + j
                pos = (dv[j] & 7) * 16
                stage[e, pl.ds(pos, 16)] = rowsb[e, :]
        pltpu.sync_copy(stage, acc_sh.at[dstb3], add=True)
        for off, j0 in ((0, 0), (16, 0), (24, 8)):
            dv = dstb[pl.ds(off, 16)]
            for j in range(j0, 16):
                e = off + j
                pos = (dv[j] & 7) * 16
                stage[e, pl.ds(pos, 16)] = jnp.zeros((16,), f32)
        return _
    lax.fori_loop(0, NCHUNK, _chunk, None)
    plsc.subcore_barrier()
    rpt2 = NPACK // NS
    pltpu.sync_copy(acc_sh.at[pl.ds(sid * rpt2, rpt2)],
                    degp_hbm.at[cid, pl.ds(sid * rpt2, rpt2)])


def _p0(dst, s16):
    mesh = plsc.VectorSubcoreMesh(core_axis_name="c", subcore_axis_name="s")
    return pl.kernel(
        _p0_body,
        out_type=jax.ShapeDtypeStruct((NC, NPACK, 128), f32),
        mesh=mesh,
        compiler_params=_SC_PARAMS,
        scratch_types=[
            pltpu.VMEM_SHARED((NPACK, 128), f32),
            pltpu.VMEM((CH,), i32),
            pltpu.VMEM((CH,), i32),
            pltpu.VMEM((CH, 16), f32),
            pltpu.VMEM((CH, 128), f32),
            pltpu.SemaphoreType.DMA,
        ],
    )(dst, s16)


def _b_body(deg_ref, lm_ref):
    d = deg_ref[0] + deg_ref[1]
    cnt = jnp.maximum(d[:, 3:4], 1.0)
    lm = d[:, 0:3] / cnt
    lm_ref[...] = lax.dot_general(jnp.eye(8, 3, dtype=f32), lm,
                                  (((1,), (1,)), ((), ())))


def _b(deg4):
    return pl.pallas_call(
        _b_body,
        grid=(NP // RB,),
        in_specs=[pl.BlockSpec((2, RB, 4), lambda j: (0, j, 0))],
        out_specs=pl.BlockSpec((8, RB), lambda j: (0, j)),
        out_shape=jax.ShapeDtypeStruct((8, NP), f32),
    )(deg4)


def _d_body(li, src_hbm, dst_hbm, s3_hbm, nsc_hbm, lm_hbm, xs_hbm,
            outp_hbm, denp_hbm, acc_sh, den_sh, asrc_t, adst_t,
            srcb, dstb, dstb3, sb, rowsb, dstage, sem):
    sid = lax.axis_index("s")
    cid = lax.axis_index("c")
    wid = sid * NC + cid
    base = sid * RPT
    rpt2 = NPACK // NS

    pltpu.sync_copy(nsc_hbm.at[pl.ds(0, NP)], asrc_t)
    pltpu.sync_copy(nsc_hbm.at[pl.ds(NP, NP)], adst_t)

    def _zrow(r, _):
        for q in range(8):
            dstage[r, pl.ds(q * 16, 16)] = jnp.zeros((16,), f32)
        return _
    lax.fori_loop(0, CH, _zrow, None)
    for k in range(RPT // CH):
        pltpu.sync_copy(dstage.at[pl.ds(0, CH)],
                        acc_sh.at[pl.ds(base + k * CH, CH)])
    for k in range(rpt2 // CH):
        pltpu.sync_copy(dstage.at[pl.ds(0, CH)],
                        den_sh.at[pl.ds(sid * rpt2 + k * CH, CH)])
    plsc.subcore_barrier()

    def _edges(c, is_self):
        nb = wid * NPW + c * CH
        if is_self:
            pltpu.sync_copy(lm_hbm.at[pl.ds(li * NP + nb, CH)], sb)
            pltpu.sync_copy(xs_hbm.at[pl.ds(nb, CH)], rowsb)
        else:
            eb = wid * EPW + c * CH
            pltpu.sync_copy(src_hbm.at[pl.ds(eb, CH)], srcb)
            pltpu.sync_copy(dst_hbm.at[pl.ds(eb, CH)], dstb)
            pltpu.sync_copy(s3_hbm.at[pl.ds(li * E + eb, CH)], sb)
            pltpu.async_copy(xs_hbm.at[srcb], rowsb, sem).wait()
        for off, j0 in ((0, 0), (16, 0), (24, 8)):
            if is_self:
                a1 = asrc_t[pl.ds(nb + off, 16)]
                a2 = adst_t[pl.ds(nb + off, 16)]
                dv = lax.iota(i32, 16) + (nb + off)
                dstb[pl.ds(off, 16)] = dv
            else:
                sv = srcb[pl.ds(off, 16)]
                dv = dstb[pl.ds(off, 16)]
                a1 = plsc.load_gather(asrc_t, [sv])
                a2 = plsc.load_gather(adst_t, [dv])
            alpha = a1 + a2 + sb[pl.ds(off, 16)]
            t = jnp.exp(jnp.maximum(alpha, 0.2 * alpha))
            dstb3[pl.ds(off, 16)] = lax.shift_right_logical(dv, 3)
            for j in range(j0, 16):
                e = off + j
                ts = t[j]
                for q in range(8):
                    rowsb[e, pl.ds(q * 16, 16)] = (
                        rowsb[e, pl.ds(q * 16, 16)] * ts)
                pos = (dv[j] & 7) * 16
                dstage[e, pl.ds(pos, 16)] = jnp.where(
                    lax.iota(i32, 16) == 0, ts, 0.0)
        pltpu.sync_copy(rowsb, acc_sh.at[dstb], add=True)
        pltpu.sync_copy(dstage, den_sh.at[dstb3], add=True)
        for off, j0 in ((0, 0), (16, 0), (24, 8)):
            if is_self:
                dv = lax.iota(i32, 16) + (nb + off)
            else:
                dv = dstb[pl.ds(off, 16)]
            for j in range(j0, 16):
                e = off + j
                pos = (dv[j] & 7) * 16
                dstage[e, pl.ds(pos, 16)] = jnp.zeros((16,), f32)

    def _chunk(c, _):
        _edges(c, False)
        return _
    lax.fori_loop(0, NCHUNK, _chunk, None)

    def _schunk(c, _):
        _edges(c, True)
        return _
    lax.fori_loop(0, NSC, _schunk, None)

    plsc.subcore_barrier()
    pltpu.sync_copy(acc_sh.at[pl.ds(base, RPT)],
                    outp_hbm.at[cid, pl.ds(base, RPT)])
    pltpu.sync_copy(den_sh.at[pl.ds(sid * rpt2, rpt2)],
                    denp_hbm.at[cid, pl.ds(sid * rpt2, rpt2)])


def _d(li, src, dst, s3, nsc, lm, xs):
    mesh = plsc.VectorSubcoreMesh(core_axis_name="c", subcore_axis_name="s")
    return pl.kernel(
        functools.partial(_d_body, li),
        out_type=[jax.ShapeDtypeStruct((NC, NP, H), f32),
                  jax.ShapeDtypeStruct((NC, NPACK, 128), f32)],
        mesh=mesh,
        compiler_params=_SC_PARAMS,
        scratch_types=[
            pltpu.VMEM_SHARED((NP, H), f32),
            pltpu.VMEM_SHARED((NPACK, 128), f32),
            pltpu.VMEM((NP,), f32),
            pltpu.VMEM((NP,), f32),
            pltpu.VMEM((CH,), i32),
            pltpu.VMEM((CH,), i32),
            pltpu.VMEM((CH,), i32),
            pltpu.VMEM((CH,), f32),
            pltpu.VMEM((CH, H), f32),
            pltpu.VMEM((CH, 128), f32),
            pltpu.SemaphoreType.DMA,
        ],
    )(src, dst, s3.reshape(-1), nsc.reshape(-1), lm.reshape(-1), xs)


def _gru(op, dp, xs, xv, wih, whh, bih, bhh, bias):
    num = op[0] + op[1]
    den = dp[0] + dp[1]
    h = jnp.maximum(num / (den + 1e-16) + bias, 0.0)
    gi = h @ wih + bih
    gh = xv @ whh + bhh
    r = jax.nn.sigmoid(gi[:, :H] + gh[:, :H])
    z = jax.nn.sigmoid(gi[:, H:2 * H] + gh[:, H:2 * H])
    ng = jnp.tanh(gi[:, 2 * H:] + r * gh[:, 2 * H:])
    return jnp.maximum((1.0 - z) * ng + z * xv, 0.0)


def _e_body(outp_ref, denp_ref, xs_ref, x_ref, wih_ref, whh_ref, bih_ref,
            bhh_ref, bias_ref, wlin_ref, att_ref,
            xn_ref, xs2_ref, nsc2_ref):
    xn = _gru(outp_ref[...], denp_ref[...], xs_ref[...], x_ref[...],
              wih_ref[...], whh_ref[...], bih_ref[...], bhh_ref[...],
              bias_ref[...])
    xs2 = xn @ wlin_ref[...]
    xn_ref[...] = xn
    xs2_ref[...] = xs2
    a = lax.dot_general(att_ref[...], xs2, (((1,), (1,)), ((), ())))
    nsc2_ref[...] = jnp.concatenate([a, jnp.zeros((6, RB), f32)], axis=0)


def _e(outp, denp, xs, x, wih, whh, bih, bhh, bias, wlin, att):
    return pl.pallas_call(
        _e_body,
        grid=(NP // RB,),
        in_specs=[
            pl.BlockSpec((2, RB, H), lambda j: (0, j, 0)),
            pl.BlockSpec((2, RB, 1), lambda j: (0, j, 0)),
            pl.BlockSpec((RB, H), lambda j: (j, 0)),
            pl.BlockSpec((RB, H), lambda j: (j, 0)),
            pl.BlockSpec((H, 3 * H), lambda j: (0, 0)),
            pl.BlockSpec((H, 3 * H), lambda j: (0, 0)),
            pl.BlockSpec((1, 3 * H), lambda j: (0, 0)),
            pl.BlockSpec((1, 3 * H), lambda j: (0, 0)),
            pl.BlockSpec((1, H), lambda j: (0, 0)),
            pl.BlockSpec((H, H), lambda j: (0, 0)),
            pl.BlockSpec((2, H), lambda j: (0, 0)),
        ],
        out_specs=[
            pl.BlockSpec((RB, H), lambda j: (j, 0)),
            pl.BlockSpec((RB, H), lambda j: (j, 0)),
            pl.BlockSpec((8, RB), lambda j: (0, j)),
        ],
        out_shape=[
            jax.ShapeDtypeStruct((NP, H), f32),
            jax.ShapeDtypeStruct((NP, H), f32),
            jax.ShapeDtypeStruct((8, NP), f32),
        ],
    )(outp, denp, xs, x, wih, whh, bih, bhh, bias, wlin, att)


def _e3_body(outp_ref, denp_ref, xs_ref, x_ref, batch_ref, wih_ref,
             whh_ref, bih_ref, bhh_ref, bias_ref, wl_ref, bl_ref,
             out_ref, acc_ref):
    j = pl.program_id(0)
    xn = _gru(outp_ref[...], denp_ref[...], xs_ref[...], x_ref[...],
              wih_ref[...], whh_ref[...], bih_ref[...], bhh_ref[...],
              bias_ref[...])
    b = batch_ref[0, 0, :]
    oh = (b[None, :] == lax.broadcasted_iota(i32, (G, RB), 0)).astype(f32)
    contrib = oh @ xn

    @pl.when(j == 0)
    def _():
        acc_ref[...] = jnp.zeros((G, H), f32)

    acc_ref[...] += contrib

    @pl.when(j == pl.num_programs(0) - 1)
    def _():
        out_ref[...] = acc_ref[...] @ wl_ref[...] + bl_ref[...]


def _e3(outp, denp, xs, x, batch3, wih, whh, bih, bhh, bias, wl, bl):
    return pl.pallas_call(
        _e3_body,
        grid=(NP // RB,),
        in_specs=[
            pl.BlockSpec((2, RB, H), lambda j: (0, j, 0)),
            pl.BlockSpec((2, RB, 1), lambda j: (0, j, 0)),
            pl.BlockSpec((RB, H), lambda j: (j, 0)),
            pl.BlockSpec((RB, H), lambda j: (j, 0)),
            pl.BlockSpec((1, 1, RB), lambda j: (j, 0, 0)),
            pl.BlockSpec((H, 3 * H), lambda j: (0, 0)),
            pl.BlockSpec((H, 3 * H), lambda j: (0, 0)),
            pl.BlockSpec((1, 3 * H), lambda j: (0, 0)),
            pl.BlockSpec((1, 3 * H), lambda j: (0, 0)),
            pl.BlockSpec((1, H), lambda j: (0, 0)),
            pl.BlockSpec((H, H), lambda j: (0, 0)),
            pl.BlockSpec((1, H), lambda j: (0, 0)),
        ],
        out_specs=pl.BlockSpec((G, H), lambda j: (0, 0)),
        out_shape=jax.ShapeDtypeStruct((G, H), f32),
        scratch_shapes=[pltpu.VMEM((G, H), f32)],
    )(outp, denp, xs, x, batch3, wih, whh, bih, bhh, bias, wl, bl)


def kernel(x, edge_index, edge_attr, batch, params):
    src = edge_index[0]
    dst = edge_index[1]
    xp = jnp.concatenate(
        [x, jnp.zeros((NP - N, x.shape[1]), f32)], axis=0)
    batchp = jnp.concatenate(
        [batch, jnp.full((NP - N,), G, jnp.int32)], axis=0)
    batch3 = batchp.reshape(NP // RB, 1, RB)

    bn = params['bn'].reshape(1, H)
    be = params['be'].reshape(1, H)
    wedge3 = jnp.concatenate(
        [params['c%d' % i]['Wedge'] for i in (1, 2, 3)], axis=0)
    attedge3 = jnp.stack(
        [params['c%d' % i]['att_edge'] for i in (1, 2, 3)], axis=0)
    att = [jnp.stack([params['c%d' % i]['att_src'],
                      params['c%d' % i]['att_dst']], axis=0)
           for i in (1, 2, 3)]
    cps = [params['c%d' % i] for i in (1, 2, 3)]
    gps = [params['g%d' % i] for i in (1, 2, 3)]

    x0, xs1, nsc1 = _a1(xp, params['Wn'], bn, cps[0]['Wlin'], att[0])
    s16, s3 = _a2(edge_attr, params['We'], be, wedge3, attedge3)
    degp = _p0(dst, s16)
    deg4 = degp.reshape(NC, NPACK, 8, 16)[..., 0:4].reshape(NC, NP, 4)
    lm = _b(deg4)

    xcur, xs, nsc = x0, xs1, nsc1
    for li in range(3):
        outp, denp = _d(li, src, dst, s3, nsc, lm, xs)
        denc = denp.reshape(NC, NPACK, 8, 16)[..., 0:1].reshape(NC, NP, 1)
        gp = gps[li]
        bih = gp['bih'].reshape(1, 3 * H)
        bhh = gp['bhh'].reshape(1, 3 * H)
        bias = cps[li]['bias'].reshape(1, H)
        if li < 2:
            xcur, xs, nsc = _e(outp, denc, xs, xcur, gp['Wih'],
                               gp['Whh'], bih, bhh, bias,
                               cps[li + 1]['Wlin'], att[li + 1])
        else:
            return _e3(outp, denc, xs, xcur, batch3, gp['Wih'],
                       gp['Whh'], bih, bhh, bias, params['Wl'],
                       params['bl'].reshape(1, H))


# speedup vs baseline: 11.8356x; 1.1291x over previous
import functools
import jax
import jax.numpy as jnp
from jax import lax
from jax.experimental import pallas as pl
from jax.experimental.pallas import tpu as pltpu
from jax.experimental.pallas import tpu_sc as plsc

N = 10000
NP = 10240
E = 320000
H = 128
G = 64
RB = 1024
EB = 1280
NC, NS = 2, 16
NW = NC * NS
EPW = E // NW
CH = 40
NCHUNK = EPW // CH
RPT = NP // NS
NPW = NP // NW
NSC = NPW // CH
NPACK = NP // 8

f32 = jnp.float32
i32 = jnp.int32

_SC_PARAMS = pltpu.CompilerParams(
    needs_layout_passes=False, use_tc_tiling_on_sc=False)


def _a1_body(x_ref, wn_ref, bn_ref, wlin_ref, att_ref,
             x0_ref, xs_ref, nsc_ref):
    x0 = jnp.maximum(x_ref[...] @ wn_ref[...] + bn_ref[...], 0.0)
    xs = x0 @ wlin_ref[...]
    x0_ref[...] = x0
    xs_ref[...] = xs
    a = lax.dot_general(att_ref[...], xs, (((1,), (1,)), ((), ())))
    nsc_ref[...] = jnp.concatenate([a, jnp.zeros((6, RB), f32)], axis=0)


def _a1(x, wn, bn, wlin, att):
    return pl.pallas_call(
        _a1_body,
        grid=(NP // RB,),
        in_specs=[
            pl.BlockSpec((RB, H), lambda j: (j, 0)),
            pl.BlockSpec((H, H), lambda j: (0, 0)),
            pl.BlockSpec((1, H), lambda j: (0, 0)),
            pl.BlockSpec((H, H), lambda j: (0, 0)),
            pl.BlockSpec((2, H), lambda j: (0, 0)),
        ],
        out_specs=[
            pl.BlockSpec((RB, H), lambda j: (j, 0)),
            pl.BlockSpec((RB, H), lambda j: (j, 0)),
            pl.BlockSpec((8, RB), lambda j: (0, j)),
        ],
        out_shape=[
            jax.ShapeDtypeStruct((NP, H), f32),
            jax.ShapeDtypeStruct((NP, H), f32),
            jax.ShapeDtypeStruct((8, NP), f32),
        ],
    )(x, wn, bn, wlin, att)


def _a2_body(ea_ref, we_ref, be_ref, wedge_ref, attedge_ref,
             s16_ref, s3_ref):
    eap = jnp.maximum(ea_ref[...] @ we_ref[...] + be_ref[...], 0.0)
    w = wedge_ref[...]
    ae = attedge_ref[...]
    v1 = w[0:H, :] @ ae[0, :][:, None]
    v2 = w[H:2 * H, :] @ ae[1, :][:, None]
    v3 = w[2 * H:, :] @ ae[2, :][:, None]
    s = eap @ jnp.concatenate([v1, v2, v3], axis=1)
    ones = jnp.ones((EB, 1), f32)
    s16_ref[...] = jnp.concatenate(
        [s, ones, jnp.zeros((EB, 12), f32)], axis=1)
    st = lax.dot_general(jnp.eye(8, 3, dtype=f32), s,
                         (((1,), (1,)), ((), ())))
    s3_ref[...] = st


def _a2(edge_attr, we, be, wedge3, attedge3):
    de = edge_attr.shape[1]
    return pl.pallas_call(
        _a2_body,
        grid=(E // EB,),
        in_specs=[
            pl.BlockSpec((EB, de), lambda j: (j, 0)),
            pl.BlockSpec((de, H), lambda j: (0, 0)),
            pl.BlockSpec((1, H), lambda j: (0, 0)),
            pl.BlockSpec((3 * H, H), lambda j: (0, 0)),
            pl.BlockSpec((3, H), lambda j: (0, 0)),
        ],
        out_specs=[
            pl.BlockSpec((EB, 16), lambda j: (j, 0)),
            pl.BlockSpec((8, EB), lambda j: (0, j)),
        ],
        out_shape=[
            jax.ShapeDtypeStruct((E, 16), f32),
            jax.ShapeDtypeStruct((8, E), f32),
        ],
    )(edge_attr, we, be, wedge3, attedge3)


def _p0_body(dst_hbm, s16_hbm, degp_hbm, acc_sh, dstb, rowsb, sem):
    sid = lax.axis_index("s")
    cid = lax.axis_index("c")
    wid = sid * NC + cid

    def _zrow(r, _):
        rowsb[r, :] = jnp.zeros((16,), f32)
        return _
    lax.fori_loop(0, CH, _zrow, None)
    base = sid * (NP // NS)
    for k in range((NP // NS) // CH):
        pltpu.sync_copy(rowsb, acc_sh.at[pl.ds(base + k * CH, CH)])
    plsc.subcore_barrier()

    def _chunk(c, _):
        eb = wid * EPW + c * CH
        pltpu.sync_copy(dst_hbm.at[pl.ds(eb, CH)], dstb)
        pltpu.sync_copy(s16_hbm.at[pl.ds(eb, CH)], rowsb)
        pltpu.sync_copy(rowsb, acc_sh.at[dstb], add=True)
        return _
    lax.fori_loop(0, NCHUNK, _chunk, None)
    plsc.subcore_barrier()
    rpt = NP // NS
    pltpu.sync_copy(acc_sh.at[pl.ds(sid * rpt, rpt)],
                    degp_hbm.at[cid, pl.ds(sid * rpt, rpt)])


def _p0(dst, s16):
    mesh = plsc.VectorSubcoreMesh(core_axis_name="c", subcore_axis_name="s")
    return pl.kernel(
        _p0_body,
        out_type=jax.ShapeDtypeStruct((NC, NP, 16), f32),
        mesh=mesh,
        compiler_params=_SC_PARAMS,
        scratch_types=[
            pltpu.VMEM_SHARED((NP, 16), f32),
            pltpu.VMEM((CH,), i32),
            pltpu.VMEM((CH, 16), f32),
            pltpu.SemaphoreType.DMA,
        ],
    )(dst, s16)


def _b_body(deg_ref, lm_ref):
    d = deg_ref[0] + deg_ref[1]
    cnt = jnp.maximum(d[:, 3:4], 1.0)
    lm = d[:, 0:3] / cnt
    lm_ref[...] = lax.dot_general(jnp.eye(8, 3, dtype=f32), lm,
                                  (((1,), (1,)), ((), ())))


def _b(deg4):
    return pl.pallas_call(
        _b_body,
        grid=(NP // RB,),
        in_specs=[pl.BlockSpec((2, RB, 4), lambda j: (0, j, 0))],
        out_specs=pl.BlockSpec((8, RB), lambda j: (0, j)),
        out_shape=jax.ShapeDtypeStruct((8, NP), f32),
    )(deg4)


def _d_body(li, src_hbm, dst_hbm, s3_hbm, nsc_hbm, lm_hbm, xs_hbm,
            outp_hbm, denp_hbm, acc_sh, den_sh, asrc_t, adst_t,
            srcb, dstb, sb, rowsb, dstage, sem):
    sid = lax.axis_index("s")
    cid = lax.axis_index("c")
    wid = sid * NC + cid
    base = sid * RPT

    pltpu.sync_copy(nsc_hbm.at[pl.ds(0, NP)], asrc_t)
    pltpu.sync_copy(nsc_hbm.at[pl.ds(NP, NP)], adst_t)

    def _zrow(r, _):
        for q in range(8):
            rowsb[r, pl.ds(q * 16, 16)] = jnp.zeros((16,), f32)
        dstage[r, :] = jnp.zeros((16,), f32)
        return _
    lax.fori_loop(0, CH, _zrow, None)
    for k in range(RPT // CH):
        pltpu.sync_copy(rowsb, acc_sh.at[pl.ds(base + k * CH, CH)])
        pltpu.sync_copy(dstage, den_sh.at[pl.ds(base + k * CH, CH)])
    plsc.subcore_barrier()

    def _edges(c, is_self):
        nb = wid * NPW + c * CH
        if is_self:
            pltpu.sync_copy(lm_hbm.at[pl.ds(li * NP + nb, CH)], sb)
            pltpu.sync_copy(xs_hbm.at[pl.ds(nb, CH)], rowsb)
        else:
            eb = wid * EPW + c * CH
            pltpu.sync_copy(src_hbm.at[pl.ds(eb, CH)], srcb)
            pltpu.sync_copy(dst_hbm.at[pl.ds(eb, CH)], dstb)
            pltpu.sync_copy(s3_hbm.at[pl.ds(li * E + eb, CH)], sb)
            pltpu.async_copy(xs_hbm.at[srcb], rowsb, sem).wait()
        for off, j0 in ((0, 0), (16, 0), (24, 8)):
            if is_self:
                a1 = asrc_t[pl.ds(nb + off, 16)]
                a2 = adst_t[pl.ds(nb + off, 16)]
                dv = lax.iota(i32, 16) + (nb + off)
                dstb[pl.ds(off, 16)] = dv
            else:
                sv = srcb[pl.ds(off, 16)]
                dv = dstb[pl.ds(off, 16)]
                a1 = plsc.load_gather(asrc_t, [sv])
                a2 = plsc.load_gather(adst_t, [dv])
            alpha = a1 + a2 + sb[pl.ds(off, 16)]
            t = jnp.exp(jnp.maximum(alpha, 0.2 * alpha))
            for j in range(j0, 16):
                e = off + j
                ts = t[j]
                for q in range(8):
                    rowsb[e, pl.ds(q * 16, 16)] = (
                        rowsb[e, pl.ds(q * 16, 16)] * ts)
                dstage[e, :] = jnp.where(lax.iota(i32, 16) == 0, ts, 0.0)
        pltpu.sync_copy(rowsb, acc_sh.at[dstb], add=True)
        pltpu.sync_copy(dstage, den_sh.at[dstb], add=True)

    def _chunk(c, _):
        _edges(c, False)
        return _
    lax.fori_loop(0, NCHUNK, _chunk, None)

    def _schunk(c, _):
        _edges(c, True)
        return _
    lax.fori_loop(0, NSC, _schunk, None)

    plsc.subcore_barrier()
    pltpu.sync_copy(acc_sh.at[pl.ds(base, RPT)],
                    outp_hbm.at[cid, pl.ds(base, RPT)])
    pltpu.sync_copy(den_sh.at[pl.ds(base, RPT)],
                    denp_hbm.at[cid, pl.ds(base, RPT)])


def _d(li, src, dst, s3, nsc, lm, xs):
    mesh = plsc.VectorSubcoreMesh(core_axis_name="c", subcore_axis_name="s")
    return pl.kernel(
        functools.partial(_d_body, li),
        out_type=[jax.ShapeDtypeStruct((NC, NP, H), f32),
                  jax.ShapeDtypeStruct((NC, NP, 16), f32)],
        mesh=mesh,
        compiler_params=_SC_PARAMS,
        scratch_types=[
            pltpu.VMEM_SHARED((NP, H), f32),
            pltpu.VMEM_SHARED((NP, 16), f32),
            pltpu.VMEM((NP,), f32),
            pltpu.VMEM((NP,), f32),
            pltpu.VMEM((CH,), i32),
            pltpu.VMEM((CH,), i32),
            pltpu.VMEM((CH,), f32),
            pltpu.VMEM((CH, H), f32),
            pltpu.VMEM((CH, 16), f32),
            pltpu.SemaphoreType.DMA,
        ],
    )(src, dst, s3.reshape(-1), nsc.reshape(-1), lm.reshape(-1), xs)


def _gru(op, dp, xs, xv, wih, whh, bih, bhh, bias):
    num = op[0] + op[1]
    den = dp[0] + dp[1]
    h = jnp.maximum(num / (den + 1e-16) + bias, 0.0)
    gi = h @ wih + bih
    gh = xv @ whh + bhh
    r = jax.nn.sigmoid(gi[:, :H] + gh[:, :H])
    z = jax.nn.sigmoid(gi[:, H:2 * H] + gh[:, H:2 * H])
    ng = jnp.tanh(gi[:, 2 * H:] + r * gh[:, 2 * H:])
    return jnp.maximum((1.0 - z) * ng + z * xv, 0.0)


def _e_body(outp_ref, denp_ref, xs_ref, x_ref, wih_ref, whh_ref, bih_ref,
            bhh_ref, bias_ref, wlin_ref, att_ref,
            xn_ref, xs2_ref, nsc2_ref):
    xn = _gru(outp_ref[...], denp_ref[...], xs_ref[...], x_ref[...],
              wih_ref[...], whh_ref[...], bih_ref[...], bhh_ref[...],
              bias_ref[...])
    xs2 = xn @ wlin_ref[...]
    xn_ref[...] = xn
    xs2_ref[...] = xs2
    a = lax.dot_general(att_ref[...], xs2, (((1,), (1,)), ((), ())))
    nsc2_ref[...] = jnp.concatenate([a, jnp.zeros((6, RB), f32)], axis=0)


def _e(outp, denp, xs, x, wih, whh, bih, bhh, bias, wlin, att):
    return pl.pallas_call(
        _e_body,
        grid=(NP // RB,),
        in_specs=[
            pl.BlockSpec((2, RB, H), lambda j: (0, j, 0)),
            pl.BlockSpec((2, RB, 1), lambda j: (0, j, 0)),
            pl.BlockSpec((RB, H), lambda j: (j, 0)),
            pl.BlockSpec((RB, H), lambda j: (j, 0)),
            pl.BlockSpec((H, 3 * H), lambda j: (0, 0)),
            pl.BlockSpec((H, 3 * H), lambda j: (0, 0)),
            pl.BlockSpec((1, 3 * H), lambda j: (0, 0)),
            pl.BlockSpec((1, 3 * H), lambda j: (0, 0)),
            pl.BlockSpec((1, H), lambda j: (0, 0)),
            pl.BlockSpec((H, H), lambda j: (0, 0)),
            pl.BlockSpec((2, H), lambda j: (0, 0)),
        ],
        out_specs=[
            pl.BlockSpec((RB, H), lambda j: (j, 0)),
            pl.BlockSpec((RB, H), lambda j: (j, 0)),
            pl.BlockSpec((8, RB), lambda j: (0, j)),
        ],
        out_shape=[
            jax.ShapeDtypeStruct((NP, H), f32),
            jax.ShapeDtypeStruct((NP, H), f32),
            jax.ShapeDtypeStruct((8, NP), f32),
        ],
    )(outp, denp, xs, x, wih, whh, bih, bhh, bias, wlin, att)


def _e3_body(outp_ref, denp_ref, xs_ref, x_ref, batch_ref, wih_ref,
             whh_ref, bih_ref, bhh_ref, bias_ref, wl_ref, bl_ref,
             out_ref, acc_ref):
    j = pl.program_id(0)
    xn = _gru(outp_ref[...], denp_ref[...], xs_ref[...], x_ref[...],
              wih_ref[...], whh_ref[...], bih_ref[...], bhh_ref[...],
              bias_ref[...])
    b = batch_ref[0, 0, :]
    oh = (b[None, :] == lax.broadcasted_iota(i32, (G, RB), 0)).astype(f32)
    contrib = oh @ xn

    @pl.when(j == 0)
    def _():
        acc_ref[...] = jnp.zeros((G, H), f32)

    acc_ref[...] += contrib

    @pl.when(j == pl.num_programs(0) - 1)
    def _():
        out_ref[...] = acc_ref[...] @ wl_ref[...] + bl_ref[...]


def _e3(outp, denp, xs, x, batch3, wih, whh, bih, bhh, bias, wl, bl):
    return pl.pallas_call(
        _e3_body,
        grid=(NP // RB,),
        in_specs=[
            pl.BlockSpec((2, RB, H), lambda j: (0, j, 0)),
            pl.BlockSpec((2, RB, 1), lambda j: (0, j, 0)),
            pl.BlockSpec((RB, H), lambda j: (j, 0)),
            pl.BlockSpec((RB, H), lambda j: (j, 0)),
            pl.BlockSpec((1, 1, RB), lambda j: (j, 0, 0)),
            pl.BlockSpec((H, 3 * H), lambda j: (0, 0)),
            pl.BlockSpec((H, 3 * H), lambda j: (0, 0)),
            pl.BlockSpec((1, 3 * H), lambda j: (0, 0)),
            pl.BlockSpec((1, 3 * H), lambda j: (0, 0)),
            pl.BlockSpec((1, H), lambda j: (0, 0)),
            pl.BlockSpec((H, H), lambda j: (0, 0)),
            pl.BlockSpec((1, H), lambda j: (0, 0)),
        ],
        out_specs=pl.BlockSpec((G, H), lambda j: (0, 0)),
        out_shape=jax.ShapeDtypeStruct((G, H), f32),
        scratch_shapes=[pltpu.VMEM((G, H), f32)],
    )(outp, denp, xs, x, batch3, wih, whh, bih, bhh, bias, wl, bl)


def kernel(x, edge_index, edge_attr, batch, params):
    src = edge_index[0]
    dst = edge_index[1]
    xp = jnp.concatenate(
        [x, jnp.zeros((NP - N, x.shape[1]), f32)], axis=0)
    batchp = jnp.concatenate(
        [batch, jnp.full((NP - N,), G, jnp.int32)], axis=0)
    batch3 = batchp.reshape(NP // RB, 1, RB)

    bn = params['bn'].reshape(1, H)
    be = params['be'].reshape(1, H)
    wedge3 = jnp.concatenate(
        [params['c%d' % i]['Wedge'] for i in (1, 2, 3)], axis=0)
    attedge3 = jnp.stack(
        [params['c%d' % i]['att_edge'] for i in (1, 2, 3)], axis=0)
    att = [jnp.stack([params['c%d' % i]['att_src'],
                      params['c%d' % i]['att_dst']], axis=0)
           for i in (1, 2, 3)]
    cps = [params['c%d' % i] for i in (1, 2, 3)]
    gps = [params['g%d' % i] for i in (1, 2, 3)]

    x0, xs1, nsc1 = _a1(xp, params['Wn'], bn, cps[0]['Wlin'], att[0])
    s16, s3 = _a2(edge_attr, params['We'], be, wedge3, attedge3)
    degp = _p0(dst, s16)
    deg4 = degp[..., 0:4]
    lm = _b(deg4)

    xcur, xs, nsc = x0, xs1, nsc1
    for li in range(3):
        outp, denp = _d(li, src, dst, s3, nsc, lm, xs)
        denc = denp[..., 0:1]
        gp = gps[li]
        bih = gp['bih'].reshape(1, 3 * H)
        bhh = gp['bhh'].reshape(1, 3 * H)
        bias = cps[li]['bias'].reshape(1, H)
        if li < 2:
            xcur, xs, nsc = _e(outp, denc, xs, xcur, gp['Wih'],
                               gp['Whh'], bih, bhh, bias,
                               cps[li + 1]['Wlin'], att[li + 1])
        else:
            return _e3(outp, denc, xs, xcur, batch3, gp['Wih'],
                       gp['Whh'], bih, bhh, bias, params['Wl'],
                       params['bl'].reshape(1, H))


# speedup vs baseline: 19.9496x; 1.6855x over previous
import functools
import jax
import jax.numpy as jnp
from jax import lax
from jax.experimental import pallas as pl
from jax.experimental.pallas import tpu as pltpu
from jax.experimental.pallas import tpu_sc as plsc

N = 10000
NP = 10240
E = 320000
H = 128
G = 64
RB = 1024
EB = 1280
NC, NS = 2, 16
NW = NC * NS
EPW = E // NW
CH = 40
NCHUNK = EPW // CH
RPT = NP // NS
NPW = NP // NW
NSC = NPW // CH
NPACK = NP // 8

f32 = jnp.float32
i32 = jnp.int32

_SC_PARAMS = pltpu.CompilerParams(
    needs_layout_passes=False, use_tc_tiling_on_sc=False)


def _a1_body(x_ref, wn_ref, bn_ref, wlin_ref, att_ref,
             x0_ref, xs_ref, nsc_ref):
    x0 = jnp.maximum(x_ref[...] @ wn_ref[...] + bn_ref[...], 0.0)
    xs = x0 @ wlin_ref[...]
    x0_ref[...] = x0
    xs_ref[...] = xs
    a = lax.dot_general(att_ref[...], xs, (((1,), (1,)), ((), ())))
    nsc_ref[...] = jnp.concatenate([a, jnp.zeros((6, RB), f32)], axis=0)


def _a1(x, wn, bn, wlin, att):
    return pl.pallas_call(
        _a1_body,
        grid=(NP // RB,),
        in_specs=[
            pl.BlockSpec((RB, H), lambda j: (j, 0)),
            pl.BlockSpec((H, H), lambda j: (0, 0)),
            pl.BlockSpec((1, H), lambda j: (0, 0)),
            pl.BlockSpec((H, H), lambda j: (0, 0)),
            pl.BlockSpec((2, H), lambda j: (0, 0)),
        ],
        out_specs=[
            pl.BlockSpec((RB, H), lambda j: (j, 0)),
            pl.BlockSpec((RB, H), lambda j: (j, 0)),
            pl.BlockSpec((8, RB), lambda j: (0, j)),
        ],
        out_shape=[
            jax.ShapeDtypeStruct((NP, H), f32),
            jax.ShapeDtypeStruct((NP, H), f32),
            jax.ShapeDtypeStruct((8, NP), f32),
        ],
    )(x, wn, bn, wlin, att)


def _a2_body(ea_ref, we_ref, be_ref, wedge_ref, attedge_ref,
             s16_ref, s3_ref):
    eap = jnp.maximum(ea_ref[...] @ we_ref[...] + be_ref[...], 0.0)
    w = wedge_ref[...]
    ae = attedge_ref[...]
    v1 = w[0:H, :] @ ae[0, :][:, None]
    v2 = w[H:2 * H, :] @ ae[1, :][:, None]
    v3 = w[2 * H:, :] @ ae[2, :][:, None]
    s = eap @ jnp.concatenate([v1, v2, v3], axis=1)
    ones = jnp.ones((EB, 1), f32)
    s16_ref[...] = jnp.concatenate(
        [s, ones, jnp.zeros((EB, 12), f32)], axis=1)
    st = lax.dot_general(jnp.eye(8, 3, dtype=f32), s,
                         (((1,), (1,)), ((), ())))
    s3_ref[...] = st


def _a2(edge_attr, we, be, wedge3, attedge3):
    de = edge_attr.shape[1]
    return pl.pallas_call(
        _a2_body,
        grid=(E // EB,),
        in_specs=[
            pl.BlockSpec((EB, de), lambda j: (j, 0)),
            pl.BlockSpec((de, H), lambda j: (0, 0)),
            pl.BlockSpec((1, H), lambda j: (0, 0)),
            pl.BlockSpec((3 * H, H), lambda j: (0, 0)),
            pl.BlockSpec((3, H), lambda j: (0, 0)),
        ],
        out_specs=[
            pl.BlockSpec((EB, 16), lambda j: (j, 0)),
            pl.BlockSpec((8, EB), lambda j: (0, j)),
        ],
        out_shape=[
            jax.ShapeDtypeStruct((E, 16), f32),
            jax.ShapeDtypeStruct((8, E), f32),
        ],
    )(edge_attr, we, be, wedge3, attedge3)


def _p0_body(dst_hbm, s16_hbm, degp_hbm, acc_sh, dstb, rowsb, sem):
    sid = lax.axis_index("s")
    cid = lax.axis_index("c")
    wid = sid * NC + cid

    def _zrow(r, _):
        rowsb[r, :] = jnp.zeros((16,), f32)
        return _
    lax.fori_loop(0, CH, _zrow, None)
    base = sid * (NP // NS)
    for k in range((NP // NS) // CH):
        pltpu.sync_copy(rowsb, acc_sh.at[pl.ds(base + k * CH, CH)])
    plsc.subcore_barrier()

    def _chunk(c, _):
        eb = wid * EPW + c * CH
        pltpu.sync_copy(dst_hbm.at[pl.ds(eb, CH)], dstb)
        pltpu.sync_copy(s16_hbm.at[pl.ds(eb, CH)], rowsb)
        pltpu.sync_copy(rowsb, acc_sh.at[dstb], add=True)
        return _
    lax.fori_loop(0, NCHUNK, _chunk, None)
    plsc.subcore_barrier()
    rpt = NP // NS
    pltpu.sync_copy(acc_sh.at[pl.ds(sid * rpt, rpt)],
                    degp_hbm.at[cid, pl.ds(sid * rpt, rpt)])


def _p0(dst, s16):
    mesh = plsc.VectorSubcoreMesh(core_axis_name="c", subcore_axis_name="s")
    return pl.kernel(
        _p0_body,
        out_type=jax.ShapeDtypeStruct((NC, NP, 16), f32),
        mesh=mesh,
        compiler_params=_SC_PARAMS,
        scratch_types=[
            pltpu.VMEM_SHARED((NP, 16), f32),
            pltpu.VMEM((CH,), i32),
            pltpu.VMEM((CH, 16), f32),
            pltpu.SemaphoreType.DMA,
        ],
    )(dst, s16)


def _b_body(deg_ref, lm_ref):
    d = deg_ref[0] + deg_ref[1]
    cnt = jnp.maximum(d[:, 3:4], 1.0)
    lm = d[:, 0:3] / cnt
    lm_ref[...] = lax.dot_general(jnp.eye(8, 3, dtype=f32), lm,
                                  (((1,), (1,)), ((), ())))


def _b(deg4):
    return pl.pallas_call(
        _b_body,
        grid=(NP // RB,),
        in_specs=[pl.BlockSpec((2, RB, 4), lambda j: (0, j, 0))],
        out_specs=pl.BlockSpec((8, RB), lambda j: (0, j)),
        out_shape=jax.ShapeDtypeStruct((8, NP), f32),
    )(deg4)


def _d_body(li, src_hbm, dst_hbm, s3_hbm, nsc_hbm, lm_hbm, xs_hbm,
            outp_hbm, denp_hbm, acc_sh, den_sh, asrc_t, adst_t,
            srcb0, srcb1, dstb0, dstb1, sb0, sb1, rowsb0, rowsb1,
            dstage, sem, ssem0, ssem1, gsem0, gsem1):
    sid = lax.axis_index("s")
    cid = lax.axis_index("c")
    wid = sid * NC + cid
    base = sid * RPT
    srcb = (srcb0, srcb1)
    dstb = (dstb0, dstb1)
    sb = (sb0, sb1)
    rowsb = (rowsb0, rowsb1)
    ssem = (ssem0, ssem1)
    gsem = (gsem0, gsem1)

    pltpu.sync_copy(nsc_hbm.at[pl.ds(0, NP)], asrc_t)
    pltpu.sync_copy(nsc_hbm.at[pl.ds(NP, NP)], adst_t)

    def _zrow(r, _):
        for q in range(8):
            rowsb0[r, pl.ds(q * 16, 16)] = jnp.zeros((16,), f32)
        dstage[r, :] = jnp.zeros((16,), f32)
        return _
    lax.fori_loop(0, CH, _zrow, None)
    for k in range(RPT // CH):
        pltpu.sync_copy(rowsb0, acc_sh.at[pl.ds(base + k * CH, CH)])
        pltpu.sync_copy(dstage, den_sh.at[pl.ds(base + k * CH, CH)])
    plsc.subcore_barrier()

    def _eb(c):
        cc = jnp.minimum(c, NCHUNK - 1)
        return wid * EPW + cc * CH

    def _fire_scalars(c, bi):
        eb = _eb(c)
        pltpu.async_copy(src_hbm.at[pl.ds(eb, CH)], srcb[bi], ssem[bi])
        pltpu.async_copy(dst_hbm.at[pl.ds(eb, CH)], dstb[bi], ssem[bi])
        pltpu.async_copy(s3_hbm.at[pl.ds(li * E + eb, CH)], sb[bi],
                         ssem[bi])

    def _drain_scalars(c, bi):
        eb = _eb(c)
        pltpu.make_async_copy(src_hbm.at[pl.ds(eb, CH)], srcb[bi],
                              ssem[bi]).wait()
        pltpu.make_async_copy(dst_hbm.at[pl.ds(eb, CH)], dstb[bi],
                              ssem[bi]).wait()
        pltpu.make_async_copy(s3_hbm.at[pl.ds(li * E + eb, CH)], sb[bi],
                              ssem[bi]).wait()

    def _fire_gather(bi):
        pltpu.async_copy(xs_hbm.at[srcb[bi]], rowsb[bi], gsem[bi])

    def _wait_gather(bi):
        pltpu.make_async_copy(xs_hbm.at[srcb[bi]], rowsb[bi],
                              gsem[bi]).wait()

    def _compute(bi):
        for off, j0 in ((0, 0), (16, 0), (24, 8)):
            sv = srcb[bi][pl.ds(off, 16)]
            dv = dstb[bi][pl.ds(off, 16)]
            a1 = plsc.load_gather(asrc_t, [sv])
            a2 = plsc.load_gather(adst_t, [dv])
            alpha = a1 + a2 + sb[bi][pl.ds(off, 16)]
            t = jnp.exp(jnp.maximum(alpha, 0.2 * alpha))
            for j in range(j0, 16):
                e = off + j
                ts = t[j]
                for q in range(8):
                    rowsb[bi][e, pl.ds(q * 16, 16)] = (
                        rowsb[bi][e, pl.ds(q * 16, 16)] * ts)
                dstage[e, :] = jnp.where(lax.iota(i32, 16) == 0, ts, 0.0)
        pltpu.sync_copy(rowsb[bi], acc_sh.at[dstb[bi]], add=True)
        pltpu.sync_copy(dstage, den_sh.at[dstb[bi]], add=True)

    _fire_scalars(0, 0)
    _drain_scalars(0, 0)
    _fire_gather(0)
    _fire_scalars(1, 1)

    def _body(k, _):
        c = 2 * k
        _wait_gather(0)
        _drain_scalars(c + 1, 1)
        _fire_gather(1)
        _compute(0)
        _fire_scalars(c + 2, 0)
        _wait_gather(1)
        _drain_scalars(c + 2, 0)
        _fire_gather(0)
        _compute(1)
        _fire_scalars(c + 3, 1)
        return _
    lax.fori_loop(0, NCHUNK // 2, _body, None)
    _wait_gather(0)
    _drain_scalars(NCHUNK - 1, 1)

    def _schunk(c, _):
        nb = wid * NPW + c * CH
        pltpu.sync_copy(lm_hbm.at[pl.ds(li * NP + nb, CH)], sb0)
        pltpu.sync_copy(xs_hbm.at[pl.ds(nb, CH)], rowsb0)
        for off, j0 in ((0, 0), (16, 0), (24, 8)):
            a1 = asrc_t[pl.ds(nb + off, 16)]
            a2 = adst_t[pl.ds(nb + off, 16)]
            dv = lax.iota(i32, 16) + (nb + off)
            dstb0[pl.ds(off, 16)] = dv
            alpha = a1 + a2 + sb0[pl.ds(off, 16)]
            t = jnp.exp(jnp.maximum(alpha, 0.2 * alpha))
            for j in range(j0, 16):
                e = off + j
                ts = t[j]
                for q in range(8):
                    rowsb0[e, pl.ds(q * 16, 16)] = (
                        rowsb0[e, pl.ds(q * 16, 16)] * ts)
                dstage[e, :] = jnp.where(lax.iota(i32, 16) == 0, ts, 0.0)
        pltpu.sync_copy(rowsb0, acc_sh.at[dstb0], add=True)
        pltpu.sync_copy(dstage, den_sh.at[dstb0], add=True)
        return _
    lax.fori_loop(0, NSC, _schunk, None)

    plsc.subcore_barrier()
    pltpu.sync_copy(acc_sh.at[pl.ds(base, RPT)],
                    outp_hbm.at[cid, pl.ds(base, RPT)])
    pltpu.sync_copy(den_sh.at[pl.ds(base, RPT)],
                    denp_hbm.at[cid, pl.ds(base, RPT)])


def _d(li, src, dst, s3, nsc, lm, xs):
    mesh = plsc.VectorSubcoreMesh(core_axis_name="c", subcore_axis_name="s")
    return pl.kernel(
        functools.partial(_d_body, li),
        out_type=[jax.ShapeDtypeStruct((NC, NP, H), f32),
                  jax.ShapeDtypeStruct((NC, NP, 16), f32)],
        mesh=mesh,
        compiler_params=_SC_PARAMS,
        scratch_types=[
            pltpu.VMEM_SHARED((NP, H), f32),
            pltpu.VMEM_SHARED((NP, 16), f32),
            pltpu.VMEM((NP,), f32),
            pltpu.VMEM((NP,), f32),
            pltpu.VMEM((CH,), i32),
            pltpu.VMEM((CH,), i32),
            pltpu.VMEM((CH,), i32),
            pltpu.VMEM((CH,), i32),
            pltpu.VMEM((CH,), f32),
            pltpu.VMEM((CH,), f32),
            pltpu.VMEM((CH, H), f32),
            pltpu.VMEM((CH, H), f32),
            pltpu.VMEM((CH, 16), f32),
            pltpu.SemaphoreType.DMA,
            pltpu.SemaphoreType.DMA,
            pltpu.SemaphoreType.DMA,
            pltpu.SemaphoreType.DMA,
            pltpu.SemaphoreType.DMA,
        ],
    )(src, dst, s3.reshape(-1), nsc.reshape(-1), lm.reshape(-1), xs)


def _gru(op, dp, xs, xv, wih, whh, bih, bhh, bias):
    num = op[0] + op[1]
    den = dp[0] + dp[1]
    h = jnp.maximum(num / (den + 1e-16) + bias, 0.0)
    gi = h @ wih + bih
    gh = xv @ whh + bhh
    r = jax.nn.sigmoid(gi[:, :H] + gh[:, :H])
    z = jax.nn.sigmoid(gi[:, H:2 * H] + gh[:, H:2 * H])
    ng = jnp.tanh(gi[:, 2 * H:] + r * gh[:, 2 * H:])
    return jnp.maximum((1.0 - z) * ng + z * xv, 0.0)


def _e_body(outp_ref, denp_ref, xs_ref, x_ref, wih_ref, whh_ref, bih_ref,
            bhh_ref, bias_ref, wlin_ref, att_ref,
            xn_ref, xs2_ref, nsc2_ref):
    xn = _gru(outp_ref[...], denp_ref[...], xs_ref[...], x_ref[...],
              wih_ref[...], whh_ref[...], bih_ref[...], bhh_ref[...],
              bias_ref[...])
    xs2 = xn @ wlin_ref[...]
    xn_ref[...] = xn
    xs2_ref[...] = xs2
    a = lax.dot_general(att_ref[...], xs2, (((1,), (1,)), ((), ())))
    nsc2_ref[...] = jnp.concatenate([a, jnp.zeros((6, RB), f32)], axis=0)


def _e(outp, denp, xs, x, wih, whh, bih, bhh, bias, wlin, att):
    return pl.pallas_call(
        _e_body,
        grid=(NP // RB,),
        in_specs=[
            pl.BlockSpec((2, RB, H), lambda j: (0, j, 0)),
            pl.BlockSpec((2, RB, 1), lambda j: (0, j, 0)),
            pl.BlockSpec((RB, H), lambda j: (j, 0)),
            pl.BlockSpec((RB, H), lambda j: (j, 0)),
            pl.BlockSpec((H, 3 * H), lambda j: (0, 0)),
            pl.BlockSpec((H, 3 * H), lambda j: (0, 0)),
            pl.BlockSpec((1, 3 * H), lambda j: (0, 0)),
            pl.BlockSpec((1, 3 * H), lambda j: (0, 0)),
            pl.BlockSpec((1, H), lambda j: (0, 0)),
            pl.BlockSpec((H, H), lambda j: (0, 0)),
            pl.BlockSpec((2, H), lambda j: (0, 0)),
        ],
        out_specs=[
            pl.BlockSpec((RB, H), lambda j: (j, 0)),
            pl.BlockSpec((RB, H), lambda j: (j, 0)),
            pl.BlockSpec((8, RB), lambda j: (0, j)),
        ],
        out_shape=[
            jax.ShapeDtypeStruct((NP, H), f32),
            jax.ShapeDtypeStruct((NP, H), f32),
            jax.ShapeDtypeStruct((8, NP), f32),
        ],
    )(outp, denp, xs, x, wih, whh, bih, bhh, bias, wlin, att)


def _e3_body(outp_ref, denp_ref, xs_ref, x_ref, batch_ref, wih_ref,
             whh_ref, bih_ref, bhh_ref, bias_ref, wl_ref, bl_ref,
             out_ref, acc_ref):
    j = pl.program_id(0)
    xn = _gru(outp_ref[...], denp_ref[...], xs_ref[...], x_ref[...],
              wih_ref[...], whh_ref[...], bih_ref[...], bhh_ref[...],
              bias_ref[...])
    b = batch_ref[0, 0, :]
    oh = (b[None, :] == lax.broadcasted_iota(i32, (G, RB), 0)).astype(f32)
    contrib = oh @ xn

    @pl.when(j == 0)
    def _():
        acc_ref[...] = jnp.zeros((G, H), f32)

    acc_ref[...] += contrib

    @pl.when(j == pl.num_programs(0) - 1)
    def _():
        out_ref[...] = acc_ref[...] @ wl_ref[...] + bl_ref[...]


def _e3(outp, denp, xs, x, batch3, wih, whh, bih, bhh, bias, wl, bl):
    return pl.pallas_call(
        _e3_body,
        grid=(NP // RB,),
        in_specs=[
            pl.BlockSpec((2, RB, H), lambda j: (0, j, 0)),
            pl.BlockSpec((2, RB, 1), lambda j: (0, j, 0)),
            pl.BlockSpec((RB, H), lambda j: (j, 0)),
            pl.BlockSpec((RB, H), lambda j: (j, 0)),
            pl.BlockSpec((1, 1, RB), lambda j: (j, 0, 0)),
            pl.BlockSpec((H, 3 * H), lambda j: (0, 0)),
            pl.BlockSpec((H, 3 * H), lambda j: (0, 0)),
            pl.BlockSpec((1, 3 * H), lambda j: (0, 0)),
            pl.BlockSpec((1, 3 * H), lambda j: (0, 0)),
            pl.BlockSpec((1, H), lambda j: (0, 0)),
            pl.BlockSpec((H, H), lambda j: (0, 0)),
            pl.BlockSpec((1, H), lambda j: (0, 0)),
        ],
        out_specs=pl.BlockSpec((G, H), lambda j: (0, 0)),
        out_shape=jax.ShapeDtypeStruct((G, H), f32),
        scratch_shapes=[pltpu.VMEM((G, H), f32)],
    )(outp, denp, xs, x, batch3, wih, whh, bih, bhh, bias, wl, bl)


def kernel(x, edge_index, edge_attr, batch, params):
    src = edge_index[0]
    dst = edge_index[1]
    xp = jnp.concatenate(
        [x, jnp.zeros((NP - N, x.shape[1]), f32)], axis=0)
    batchp = jnp.concatenate(
        [batch, jnp.full((NP - N,), G, jnp.int32)], axis=0)
    batch3 = batchp.reshape(NP // RB, 1, RB)

    bn = params['bn'].reshape(1, H)
    be = params['be'].reshape(1, H)
    wedge3 = jnp.concatenate(
        [params['c%d' % i]['Wedge'] for i in (1, 2, 3)], axis=0)
    attedge3 = jnp.stack(
        [params['c%d' % i]['att_edge'] for i in (1, 2, 3)], axis=0)
    att = [jnp.stack([params['c%d' % i]['att_src'],
                      params['c%d' % i]['att_dst']], axis=0)
           for i in (1, 2, 3)]
    cps = [params['c%d' % i] for i in (1, 2, 3)]
    gps = [params['g%d' % i] for i in (1, 2, 3)]

    x0, xs1, nsc1 = _a1(xp, params['Wn'], bn, cps[0]['Wlin'], att[0])
    s16, s3 = _a2(edge_attr, params['We'], be, wedge3, attedge3)
    degp = _p0(dst, s16)
    deg4 = degp[..., 0:4]
    lm = _b(deg4)

    xcur, xs, nsc = x0, xs1, nsc1
    for li in range(3):
        outp, denp = _d(li, src, dst, s3, nsc, lm, xs)
        denc = denp[..., 0:1]
        gp = gps[li]
        bih = gp['bih'].reshape(1, 3 * H)
        bhh = gp['bhh'].reshape(1, 3 * H)
        bias = cps[li]['bias'].reshape(1, H)
        if li < 2:
            xcur, xs, nsc = _e(outp, denc, xs, xcur, gp['Wih'],
                               gp['Whh'], bih, bhh, bias,
                               cps[li + 1]['Wlin'], att[li + 1])
        else:
            return _e3(outp, denc, xs, xcur, batch3, gp['Wih'],
                       gp['Whh'], bih, bhh, bias, params['Wl'],
                       params['bl'].reshape(1, H))


# speedup vs baseline: 21.6731x; 1.0864x over previous
import functools
import jax
import jax.numpy as jnp
from jax import lax
from jax.experimental import pallas as pl
from jax.experimental.pallas import tpu as pltpu
from jax.experimental.pallas import tpu_sc as plsc

N = 10000
NP = 10240
E = 320000
H = 128
G = 64
RB = 1024
EB = 1280
NC, NS = 2, 16
NW = NC * NS
EPW = E // NW
CH = 40
NCHUNK = EPW // CH
RPT = NP // NS
NPW = NP // NW
NSC = NPW // CH
NPACK = NP // 8

f32 = jnp.float32
i32 = jnp.int32

_SC_PARAMS = pltpu.CompilerParams(
    needs_layout_passes=False, use_tc_tiling_on_sc=False)


def _a1_body(x_ref, wn_ref, bn_ref, wlin_ref, att_ref,
             x0_ref, xs_ref, nsc_ref):
    x0 = jnp.maximum(x_ref[...] @ wn_ref[...] + bn_ref[...], 0.0)
    xs = x0 @ wlin_ref[...]
    x0_ref[...] = x0
    xs_ref[...] = xs
    a = lax.dot_general(att_ref[...], xs, (((1,), (1,)), ((), ())))
    nsc_ref[...] = jnp.concatenate([a, jnp.zeros((6, RB), f32)], axis=0)


def _a1(x, wn, bn, wlin, att):
    return pl.pallas_call(
        _a1_body,
        grid=(NP // RB,),
        in_specs=[
            pl.BlockSpec((RB, H), lambda j: (j, 0)),
            pl.BlockSpec((H, H), lambda j: (0, 0)),
            pl.BlockSpec((1, H), lambda j: (0, 0)),
            pl.BlockSpec((H, H), lambda j: (0, 0)),
            pl.BlockSpec((2, H), lambda j: (0, 0)),
        ],
        out_specs=[
            pl.BlockSpec((RB, H), lambda j: (j, 0)),
            pl.BlockSpec((RB, H), lambda j: (j, 0)),
            pl.BlockSpec((8, RB), lambda j: (0, j)),
        ],
        out_shape=[
            jax.ShapeDtypeStruct((NP, H), f32),
            jax.ShapeDtypeStruct((NP, H), f32),
            jax.ShapeDtypeStruct((8, NP), f32),
        ],
    )(x, wn, bn, wlin, att)


def _a2_body(ea_ref, we_ref, be_ref, wedge_ref, attedge_ref,
             s16_ref, s3_ref):
    eap = jnp.maximum(ea_ref[...] @ we_ref[...] + be_ref[...], 0.0)
    w = wedge_ref[...]
    ae = attedge_ref[...]
    v1 = w[0:H, :] @ ae[0, :][:, None]
    v2 = w[H:2 * H, :] @ ae[1, :][:, None]
    v3 = w[2 * H:, :] @ ae[2, :][:, None]
    s = eap @ jnp.concatenate([v1, v2, v3], axis=1)
    ones = jnp.ones((EB, 1), f32)
    s16_ref[...] = jnp.concatenate(
        [s, ones, jnp.zeros((EB, 12), f32)], axis=1)
    st = lax.dot_general(jnp.eye(8, 3, dtype=f32), s,
                         (((1,), (1,)), ((), ())))
    s3_ref[...] = st


def _a2(edge_attr, we, be, wedge3, attedge3):
    de = edge_attr.shape[1]
    return pl.pallas_call(
        _a2_body,
        grid=(E // EB,),
        in_specs=[
            pl.BlockSpec((EB, de), lambda j: (j, 0)),
            pl.BlockSpec((de, H), lambda j: (0, 0)),
            pl.BlockSpec((1, H), lambda j: (0, 0)),
            pl.BlockSpec((3 * H, H), lambda j: (0, 0)),
            pl.BlockSpec((3, H), lambda j: (0, 0)),
        ],
        out_specs=[
            pl.BlockSpec((EB, 16), lambda j: (j, 0)),
            pl.BlockSpec((8, EB), lambda j: (0, j)),
        ],
        out_shape=[
            jax.ShapeDtypeStruct((E, 16), f32),
            jax.ShapeDtypeStruct((8, E), f32),
        ],
    )(edge_attr, we, be, wedge3, attedge3)


def _p0_body(dst_hbm, s16_hbm, degp_hbm, acc_sh, dstb, rowsb, sem):
    sid = lax.axis_index("s")
    cid = lax.axis_index("c")
    wid = sid * NC + cid

    def _zrow(r, _):
        rowsb[r, :] = jnp.zeros((16,), f32)
        return _
    lax.fori_loop(0, CH, _zrow, None)
    base = sid * (NP // NS)
    for k in range((NP // NS) // CH):
        pltpu.sync_copy(rowsb, acc_sh.at[pl.ds(base + k * CH, CH)])
    plsc.subcore_barrier()

    def _chunk(c, _):
        eb = wid * EPW + c * CH
        pltpu.sync_copy(dst_hbm.at[pl.ds(eb, CH)], dstb)
        pltpu.sync_copy(s16_hbm.at[pl.ds(eb, CH)], rowsb)
        pltpu.sync_copy(rowsb, acc_sh.at[dstb], add=True)
        return _
    lax.fori_loop(0, NCHUNK, _chunk, None)
    plsc.subcore_barrier()
    rpt = NP // NS
    pltpu.sync_copy(acc_sh.at[pl.ds(sid * rpt, rpt)],
                    degp_hbm.at[cid, pl.ds(sid * rpt, rpt)])


def _p0(dst, s16):
    mesh = plsc.VectorSubcoreMesh(core_axis_name="c", subcore_axis_name="s")
    return pl.kernel(
        _p0_body,
        out_type=jax.ShapeDtypeStruct((NC, NP, 16), f32),
        mesh=mesh,
        compiler_params=_SC_PARAMS,
        scratch_types=[
            pltpu.VMEM_SHARED((NP, 16), f32),
            pltpu.VMEM((CH,), i32),
            pltpu.VMEM((CH, 16), f32),
            pltpu.SemaphoreType.DMA,
        ],
    )(dst, s16)


def _b_body(deg_ref, lm_ref):
    d = deg_ref[0] + deg_ref[1]
    cnt = jnp.maximum(d[:, 3:4], 1.0)
    lm = d[:, 0:3] / cnt
    lm_ref[...] = lax.dot_general(jnp.eye(8, 3, dtype=f32), lm,
                                  (((1,), (1,)), ((), ())))


def _b(deg4):
    return pl.pallas_call(
        _b_body,
        grid=(NP // RB,),
        in_specs=[pl.BlockSpec((2, RB, 4), lambda j: (0, j, 0))],
        out_specs=pl.BlockSpec((8, RB), lambda j: (0, j)),
        out_shape=jax.ShapeDtypeStruct((8, NP), f32),
    )(deg4)


def _d_body(li, src_hbm, dst_hbm, s3_hbm, nsc_hbm, lm_hbm, xs_hbm,
            outp_hbm, denp_hbm, acc_sh, den_sh, asrc_t, adst_t,
            srcb0, srcb1, dstb0, dstb1, sb0, sb1, rowsb0, rowsb1,
            dstage0, dstage1, sem, ssem0, ssem1, gsem0, gsem1,
            csem0, csem1):
    sid = lax.axis_index("s")
    cid = lax.axis_index("c")
    wid = sid * NC + cid
    base = sid * RPT
    srcb = (srcb0, srcb1)
    dstb = (dstb0, dstb1)
    sb = (sb0, sb1)
    rowsb = (rowsb0, rowsb1)
    dstage = (dstage0, dstage1)
    ssem = (ssem0, ssem1)
    gsem = (gsem0, gsem1)
    csem = (csem0, csem1)

    pltpu.sync_copy(nsc_hbm.at[pl.ds(0, NP)], asrc_t)
    pltpu.sync_copy(nsc_hbm.at[pl.ds(NP, NP)], adst_t)

    def _zrow(r, _):
        for q in range(8):
            rowsb0[r, pl.ds(q * 16, 16)] = jnp.zeros((16,), f32)
            rowsb1[r, pl.ds(q * 16, 16)] = jnp.zeros((16,), f32)
        dstage0[r, :] = jnp.zeros((16,), f32)
        dstage1[r, :] = jnp.zeros((16,), f32)
        return _
    lax.fori_loop(0, CH, _zrow, None)

    for off in (0, 16, 24):
        dstb1[pl.ds(off, 16)] = jnp.zeros((16,), i32)
    for k in range(RPT // CH):
        pltpu.sync_copy(rowsb0, acc_sh.at[pl.ds(base + k * CH, CH)])
        pltpu.sync_copy(dstage0, den_sh.at[pl.ds(base + k * CH, CH)])
    plsc.subcore_barrier()

    def _eb(c):
        cc = jnp.minimum(c, NCHUNK - 1)
        return wid * EPW + cc * CH

    def _fire_scalars(c, bi):
        eb = _eb(c)
        pltpu.async_copy(src_hbm.at[pl.ds(eb, CH)], srcb[bi], ssem[bi])
        pltpu.async_copy(dst_hbm.at[pl.ds(eb, CH)], dstb[bi], ssem[bi])
        pltpu.async_copy(s3_hbm.at[pl.ds(li * E + eb, CH)], sb[bi],
                         ssem[bi])

    def _drain_scalars(c, bi):
        eb = _eb(c)
        pltpu.make_async_copy(src_hbm.at[pl.ds(eb, CH)], srcb[bi],
                              ssem[bi]).wait()
        pltpu.make_async_copy(dst_hbm.at[pl.ds(eb, CH)], dstb[bi],
                              ssem[bi]).wait()
        pltpu.make_async_copy(s3_hbm.at[pl.ds(li * E + eb, CH)], sb[bi],
                              ssem[bi]).wait()

    def _fire_gather(bi):
        pltpu.async_copy(xs_hbm.at[srcb[bi]], rowsb[bi], gsem[bi])

    def _wait_gather(bi):
        pltpu.make_async_copy(xs_hbm.at[srcb[bi]], rowsb[bi],
                              gsem[bi]).wait()

    def _compute(bi):
        for off, j0 in ((0, 0), (16, 0), (24, 8)):
            sv = srcb[bi][pl.ds(off, 16)]
            dv = dstb[bi][pl.ds(off, 16)]
            a1 = plsc.load_gather(asrc_t, [sv])
            a2 = plsc.load_gather(adst_t, [dv])
            alpha = a1 + a2 + sb[bi][pl.ds(off, 16)]
            t = jnp.exp(jnp.maximum(alpha, 0.2 * alpha))
            for j in range(j0, 16):
                e = off + j
                ts = t[j]
                for q in range(8):
                    rowsb[bi][e, pl.ds(q * 16, 16)] = (
                        rowsb[bi][e, pl.ds(q * 16, 16)] * ts)
                dstage[bi][e, :] = jnp.where(
                    lax.iota(i32, 16) == 0, ts, 0.0)
        pltpu.async_copy(rowsb[bi], acc_sh.at[dstb[bi]], csem[bi],
                         add=True)
        pltpu.async_copy(dstage[bi], den_sh.at[dstb[bi]], csem[bi],
                         add=True)

    def _drain_scatter(bi):
        pltpu.make_async_copy(rowsb[bi], acc_sh.at[dstb[bi]],
                              csem[bi]).wait()
        pltpu.make_async_copy(dstage[bi], den_sh.at[dstb[bi]],
                              csem[bi]).wait()

    _fire_scalars(0, 0)
    _drain_scalars(0, 0)
    _fire_gather(0)
    _fire_scalars(1, 1)
    pltpu.async_copy(rowsb1, acc_sh.at[dstb1], csem1, add=True)
    pltpu.async_copy(dstage1, den_sh.at[dstb1], csem1, add=True)

    def _body(k, _):
        c = 2 * k
        _wait_gather(0)
        _drain_scalars(c + 1, 1)
        _drain_scatter(1)
        _fire_gather(1)
        _compute(0)
        _fire_scalars(c + 2, 0)
        _wait_gather(1)
        _drain_scalars(c + 2, 0)
        _drain_scatter(0)
        _fire_gather(0)
        _compute(1)
        _fire_scalars(c + 3, 1)
        return _
    lax.fori_loop(0, NCHUNK // 2, _body, None)
    _wait_gather(0)
    _drain_scalars(NCHUNK - 1, 1)
    _drain_scatter(1)

    def _schunk(c, _):
        nb = wid * NPW + c * CH
        pltpu.sync_copy(lm_hbm.at[pl.ds(li * NP + nb, CH)], sb0)
        pltpu.sync_copy(xs_hbm.at[pl.ds(nb, CH)], rowsb0)
        for off, j0 in ((0, 0), (16, 0), (24, 8)):
            a1 = asrc_t[pl.ds(nb + off, 16)]
            a2 = adst_t[pl.ds(nb + off, 16)]
            dv = lax.iota(i32, 16) + (nb + off)
            dstb0[pl.ds(off, 16)] = dv
            alpha = a1 + a2 + sb0[pl.ds(off, 16)]
            t = jnp.exp(jnp.maximum(alpha, 0.2 * alpha))
            for j in range(j0, 16):
                e = off + j
                ts = t[j]
                for q in range(8):
                    rowsb0[e, pl.ds(q * 16, 16)] = (
                        rowsb0[e, pl.ds(q * 16, 16)] * ts)
                dstage0[e, :] = jnp.where(lax.iota(i32, 16) == 0, ts, 0.0)
        pltpu.sync_copy(rowsb0, acc_sh.at[dstb0], add=True)
        pltpu.sync_copy(dstage0, den_sh.at[dstb0], add=True)
        return _
    lax.fori_loop(0, NSC, _schunk, None)

    plsc.subcore_barrier()
    pltpu.sync_copy(acc_sh.at[pl.ds(base, RPT)],
                    outp_hbm.at[cid, pl.ds(base, RPT)])
    pltpu.sync_copy(den_sh.at[pl.ds(base, RPT)],
                    denp_hbm.at[cid, pl.ds(base, RPT)])


def _d(li, src, dst, s3, nsc, lm, xs):
    mesh = plsc.VectorSubcoreMesh(core_axis_name="c", subcore_axis_name="s")
    return pl.kernel(
        functools.partial(_d_body, li),
        out_type=[jax.ShapeDtypeStruct((NC, NP, H), f32),
                  jax.ShapeDtypeStruct((NC, NP, 16), f32)],
        mesh=mesh,
        compiler_params=_SC_PARAMS,
        scratch_types=[
            pltpu.VMEM_SHARED((NP, H), f32),
            pltpu.VMEM_SHARED((NP, 16), f32),
            pltpu.VMEM((NP,), f32),
            pltpu.VMEM((NP,), f32),
            pltpu.VMEM((CH,), i32),
            pltpu.VMEM((CH,), i32),
            pltpu.VMEM((CH,), i32),
            pltpu.VMEM((CH,), i32),
            pltpu.VMEM((CH,), f32),
            pltpu.VMEM((CH,), f32),
            pltpu.VMEM((CH, H), f32),
            pltpu.VMEM((CH, H), f32),
            pltpu.VMEM((CH, 16), f32),
            pltpu.VMEM((CH, 16), f32),
            pltpu.SemaphoreType.DMA,
            pltpu.SemaphoreType.DMA,
            pltpu.SemaphoreType.DMA,
            pltpu.SemaphoreType.DMA,
            pltpu.SemaphoreType.DMA,
            pltpu.SemaphoreType.DMA,
            pltpu.SemaphoreType.DMA,
        ],
    )(src, dst, s3.reshape(-1), nsc.reshape(-1), lm.reshape(-1), xs)


def _gru(op, dp, xs, xv, wih, whh, bih, bhh, bias):
    num = op[0] + op[1]
    den = dp[0] + dp[1]
    h = jnp.maximum(num / (den + 1e-16) + bias, 0.0)
    gi = h @ wih + bih
    gh = xv @ whh + bhh
    r = jax.nn.sigmoid(gi[:, :H] + gh[:, :H])
    z = jax.nn.sigmoid(gi[:, H:2 * H] + gh[:, H:2 * H])
    ng = jnp.tanh(gi[:, 2 * H:] + r * gh[:, 2 * H:])
    return jnp.maximum((1.0 - z) * ng + z * xv, 0.0)


def _e_body(outp_ref, denp_ref, xs_ref, x_ref, wih_ref, whh_ref, bih_ref,
            bhh_ref, bias_ref, wlin_ref, att_ref,
            xn_ref, xs2_ref, nsc2_ref):
    xn = _gru(outp_ref[...], denp_ref[...], xs_ref[...], x_ref[...],
              wih_ref[...], whh_ref[...], bih_ref[...], bhh_ref[...],
              bias_ref[...])
    xs2 = xn @ wlin_ref[...]
    xn_ref[...] = xn
    xs2_ref[...] = xs2
    a = lax.dot_general(att_ref[...], xs2, (((1,), (1,)), ((), ())))
    nsc2_ref[...] = jnp.concatenate([a, jnp.zeros((6, RB), f32)], axis=0)


def _e(outp, denp, xs, x, wih, whh, bih, bhh, bias, wlin, att):
    return pl.pallas_call(
        _e_body,
        grid=(NP // RB,),
        in_specs=[
            pl.BlockSpec((2, RB, H), lambda j: (0, j, 0)),
            pl.BlockSpec((2, RB, 1), lambda j: (0, j, 0)),
            pl.BlockSpec((RB, H), lambda j: (j, 0)),
            pl.BlockSpec((RB, H), lambda j: (j, 0)),
            pl.BlockSpec((H, 3 * H), lambda j: (0, 0)),
            pl.BlockSpec((H, 3 * H), lambda j: (0, 0)),
            pl.BlockSpec((1, 3 * H), lambda j: (0, 0)),
            pl.BlockSpec((1, 3 * H), lambda j: (0, 0)),
            pl.BlockSpec((1, H), lambda j: (0, 0)),
            pl.BlockSpec((H, H), lambda j: (0, 0)),
            pl.BlockSpec((2, H), lambda j: (0, 0)),
        ],
        out_specs=[
            pl.BlockSpec((RB, H), lambda j: (j, 0)),
            pl.BlockSpec((RB, H), lambda j: (j, 0)),
            pl.BlockSpec((8, RB), lambda j: (0, j)),
        ],
        out_shape=[
            jax.ShapeDtypeStruct((NP, H), f32),
            jax.ShapeDtypeStruct((NP, H), f32),
            jax.ShapeDtypeStruct((8, NP), f32),
        ],
    )(outp, denp, xs, x, wih, whh, bih, bhh, bias, wlin, att)


def _e3_body(outp_ref, denp_ref, xs_ref, x_ref, batch_ref, wih_ref,
             whh_ref, bih_ref, bhh_ref, bias_ref, wl_ref, bl_ref,
             out_ref, acc_ref):
    j = pl.program_id(0)
    xn = _gru(outp_ref[...], denp_ref[...], xs_ref[...], x_ref[...],
              wih_ref[...], whh_ref[...], bih_ref[...], bhh_ref[...],
              bias_ref[...])
    b = batch_ref[0, 0, :]
    oh = (b[None, :] == lax.broadcasted_iota(i32, (G, RB), 0)).astype(f32)
    contrib = oh @ xn

    @pl.when(j == 0)
    def _():
        acc_ref[...] = jnp.zeros((G, H), f32)

    acc_ref[...] += contrib

    @pl.when(j == pl.num_programs(0) - 1)
    def _():
        out_ref[...] = acc_ref[...] @ wl_ref[...] + bl_ref[...]


def _e3(outp, denp, xs, x, batch3, wih, whh, bih, bhh, bias, wl, bl):
    return pl.pallas_call(
        _e3_body,
        grid=(NP // RB,),
        in_specs=[
            pl.BlockSpec((2, RB, H), lambda j: (0, j, 0)),
            pl.BlockSpec((2, RB, 1), lambda j: (0, j, 0)),
            pl.BlockSpec((RB, H), lambda j: (j, 0)),
            pl.BlockSpec((RB, H), lambda j: (j, 0)),
            pl.BlockSpec((1, 1, RB), lambda j: (j, 0, 0)),
            pl.BlockSpec((H, 3 * H), lambda j: (0, 0)),
            pl.BlockSpec((H, 3 * H), lambda j: (0, 0)),
            pl.BlockSpec((1, 3 * H), lambda j: (0, 0)),
            pl.BlockSpec((1, 3 * H), lambda j: (0, 0)),
            pl.BlockSpec((1, H), lambda j: (0, 0)),
            pl.BlockSpec((H, H), lambda j: (0, 0)),
            pl.BlockSpec((1, H), lambda j: (0, 0)),
        ],
        out_specs=pl.BlockSpec((G, H), lambda j: (0, 0)),
        out_shape=jax.ShapeDtypeStruct((G, H), f32),
        scratch_shapes=[pltpu.VMEM((G, H), f32)],
    )(outp, denp, xs, x, batch3, wih, whh, bih, bhh, bias, wl, bl)


def kernel(x, edge_index, edge_attr, batch, params):
    src = edge_index[0]
    dst = edge_index[1]
    xp = jnp.concatenate(
        [x, jnp.zeros((NP - N, x.shape[1]), f32)], axis=0)
    batchp = jnp.concatenate(
        [batch, jnp.full((NP - N,), G, jnp.int32)], axis=0)
    batch3 = batchp.reshape(NP // RB, 1, RB)

    bn = params['bn'].reshape(1, H)
    be = params['be'].reshape(1, H)
    wedge3 = jnp.concatenate(
        [params['c%d' % i]['Wedge'] for i in (1, 2, 3)], axis=0)
    attedge3 = jnp.stack(
        [params['c%d' % i]['att_edge'] for i in (1, 2, 3)], axis=0)
    att = [jnp.stack([params['c%d' % i]['att_src'],
                      params['c%d' % i]['att_dst']], axis=0)
           for i in (1, 2, 3)]
    cps = [params['c%d' % i] for i in (1, 2, 3)]
    gps = [params['g%d' % i] for i in (1, 2, 3)]

    x0, xs1, nsc1 = _a1(xp, params['Wn'], bn, cps[0]['Wlin'], att[0])
    s16, s3 = _a2(edge_attr, params['We'], be, wedge3, attedge3)
    degp = _p0(dst, s16)
    deg4 = degp[..., 0:4]
    lm = _b(deg4)

    xcur, xs, nsc = x0, xs1, nsc1
    for li in range(3):
        outp, denp = _d(li, src, dst, s3, nsc, lm, xs)
        denc = denp[..., 0:1]
        gp = gps[li]
        bih = gp['bih'].reshape(1, 3 * H)
        bhh = gp['bhh'].reshape(1, 3 * H)
        bias = cps[li]['bias'].reshape(1, H)
        if li < 2:
            xcur, xs, nsc = _e(outp, denc, xs, xcur, gp['Wih'],
                               gp['Whh'], bih, bhh, bias,
                               cps[li + 1]['Wlin'], att[li + 1])
        else:
            return _e3(outp, denc, xs, xcur, batch3, gp['Wih'],
                       gp['Whh'], bih, bhh, bias, params['Wl'],
                       params['bl'].reshape(1, H))
